# baseline (speedup 1.0000x reference)
import functools

import numpy as np
import jax
import jax.numpy as jnp
from jax import lax
from jax.experimental import pallas as pl
from jax.experimental.pallas import tpu as pltpu
from jax.experimental.pallas import tpu_sc as plsc

NC = 2
NS = 16
NW = NC * NS
C = 128
HW = 16
MW = 32
EBLK = 2048


def _leaky(x):
    return jnp.where(x >= 0, x, 0.01 * x)


def _embed_body(ntb, x_ref, w0r, b0r, w1r, b1r, w2r, b2r, o_ref):
    sel = pl.program_id(0) < ntb
    w0 = jnp.where(sel, w0r[0], w0r[1])
    b0 = jnp.where(sel, b0r[0], b0r[1])
    w1 = jnp.where(sel, w1r[0], w1r[1])
    b1 = jnp.where(sel, b1r[0], b1r[1])
    w2 = jnp.where(sel, w2r[0], w2r[1])
    b2 = jnp.where(sel, b2r[0], b2r[1])
    x = x_ref[...]
    h = _leaky(jnp.dot(x, w0, preferred_element_type=jnp.float32) + b0[None, :])
    h = _leaky(jnp.dot(h, w1, preferred_element_type=jnp.float32) + b1[None, :])
    h = jnp.dot(h, w2, preferred_element_type=jnp.float32) + b2[None, :]
    o_ref[...] = h


def _embed(xs, wt, wo, blk, ntb):
    n = xs.shape[0]
    ws = [jnp.stack([a[0], b[0]]) for a, b in zip(wt, wo)]
    bs = [jnp.stack([a[1], b[1]]) for a, b in zip(wt, wo)]
    full = lambda r: pl.BlockSpec(r, lambda i: (0,) * len(r))
    args = []
    specs = [pl.BlockSpec((blk, xs.shape[1]), lambda i: (i, 0))]
    for w, b in zip(ws, bs):
        args += [w, b]
        specs += [full(w.shape), full(b.shape)]
    return pl.pallas_call(
        functools.partial(_embed_body, ntb),
        grid=(n // blk,),
        in_specs=specs,
        out_specs=pl.BlockSpec((blk, HW), lambda i: (i, 0)),
        out_shape=jax.ShapeDtypeStruct((n, HW), jnp.float32),
    )(xs, *args)


def _make_gather(e, k):
    pt = e // NW
    chunks = pt // C
    mesh = plsc.VectorSubcoreMesh(core_axis_name="c", subcore_axis_name="s")

    @functools.partial(
        pl.kernel,
        out_type=jax.ShapeDtypeStruct((e, HW), jnp.float32),
        mesh=mesh,
        compiler_params=pltpu.CompilerParams(use_tc_tiling_on_sc=False),
        scratch_types=[
            pltpu.VMEM((chunks, C), jnp.int32),
            pltpu.VMEM((k, C, HW), jnp.float32),
            pltpu.SemaphoreType.DMA,
            pltpu.SemaphoreType.DMA,
        ],
    )
    def gather(h_hbm, idx_hbm, out_hbm, idx_v, bufs, gsem, osem):
        wid = lax.axis_index("s") * NC + lax.axis_index("c")
        pltpu.sync_copy(idx_hbm.at[wid], idx_v)

        def grp(g, carry):
            cps = [
                pltpu.async_copy(h_hbm.at[idx_v.at[g * k + t]], bufs.at[t], gsem)
                for t in range(k)
            ]
            for cp in cps:
                cp.wait()
            ocs = [
                pltpu.async_copy(
                    bufs.at[t],
                    out_hbm.at[pl.ds((wid * chunks + g * k + t) * C, C)],
                    osem,
                )
                for t in range(k)
            ]
            for oc in ocs:
                oc.wait()
            return carry

        lax.fori_loop(0, chunks // k, grp, 0)

    return gather


def _msg_body(blk, eft_ref, xjp_ref, w1t, b1c, w2t, b2c, t2hi, t2lo, b3t,
              o_ref):
    m = blk // 8
    ef_t = eft_ref[...]
    xjT = xjp_ref[...].T
    h1 = _leaky(jnp.dot(w1t[...], ef_t, preferred_element_type=jnp.float32)
                + b1c[...])
    h2 = _leaky(jnp.dot(w2t[...], h1, preferred_element_type=jnp.float32)
                + b2c[...])
    msg_t = []
    for t in range(8):
        xjt = xjT[t * HW:(t + 1) * HW]
        h2t = h2[:, t * m:(t + 1) * m]
        z = jnp.concatenate([xjt[i:i + 1, :] * h2t for i in range(HW)], axis=0)
        zhi = z.astype(jnp.bfloat16).astype(jnp.float32)
        zlo = z - zhi
        mt = (jnp.dot(t2hi[...], zhi, preferred_element_type=jnp.float32)
              + jnp.dot(t2hi[...], zlo, preferred_element_type=jnp.float32)
              + jnp.dot(t2lo[...], zhi, preferred_element_type=jnp.float32)
              + jnp.dot(b3t[...], xjt, preferred_element_type=jnp.float32))
        msg_t.append(mt)
    cnt = jnp.concatenate(
        [jnp.ones((1, m), jnp.float32), jnp.zeros((HW - 1, m), jnp.float32)],
        axis=0)
    low = jnp.concatenate(
        [msg_t[0], cnt, msg_t[2], cnt, msg_t[4], cnt, msg_t[6], cnt], axis=0)
    high = jnp.concatenate(
        [msg_t[1], cnt, msg_t[3], cnt, msg_t[5], cnt, msg_t[7], cnt], axis=0)
    o_ref[...] = jnp.concatenate([low, high], axis=1).T


def _edge_consts(edge_ps):
    (w1, b1), (w2, b2), (w3, b3) = edge_ps
    hdim = w3.shape[0]
    out_c = 16
    in_c = w3.shape[1] // out_c
    t2 = w3.reshape(hdim, in_c, out_c).transpose(1, 0, 2).reshape(in_c * hdim, out_c)
    b3r = b3.reshape(in_c, out_c)
    t2t = t2.T
    t2hi = t2t.astype(jnp.bfloat16).astype(jnp.float32)
    t2lo = t2t - t2hi
    return (w1.T, b1.reshape(-1, 1), w2.T, b2.reshape(-1, 1), t2hi, t2lo,
            b3r.T)


def _msg(eft, xjp, consts, blk):
    e = eft.shape[1]
    full = lambda r: pl.BlockSpec(r, lambda i: (0,) * len(r))
    specs = [pl.BlockSpec((16, blk), lambda i: (0, i)),
             pl.BlockSpec((blk // 8, 128), lambda i: (i, 0))]
    specs += [full(c.shape) for c in consts]
    return pl.pallas_call(
        functools.partial(_msg_body, blk),
        grid=(e // blk,),
        in_specs=specs,
        out_specs=pl.BlockSpec((blk // 4, 128), lambda i: (i, 0)),
        out_shape=jax.ShapeDtypeStruct((e // 4, 128), jnp.float32),
    )(eft, xjp, *consts)


def _make_scatter(e, nd, k):
    pt = e // NW
    chunks = pt // C
    bufrows = k * C
    mesh = plsc.VectorSubcoreMesh(core_axis_name="c", subcore_axis_name="s")

    @functools.partial(
        pl.kernel,
        out_type=jax.ShapeDtypeStruct((NC, nd, MW), jnp.float32),
        mesh=mesh,
        compiler_params=pltpu.CompilerParams(use_tc_tiling_on_sc=False),
        scratch_types=[
            pltpu.VMEM((chunks, C), jnp.int32),
            pltpu.VMEM((bufrows, MW), jnp.float32),
            pltpu.VMEM_SHARED((nd, MW), jnp.float32),
            pltpu.SemaphoreType.DMA,
        ],
    )
    def scatter(msg_hbm, idx_hbm, zer_hbm, out_hbm, idx_v, buf, acc, sem):
        cid = lax.axis_index("c")
        sid = lax.axis_index("s")
        wid = sid * NC + cid

        @pl.when(sid == 0)
        def _():
            pltpu.sync_copy(zer_hbm, acc)

        plsc.subcore_barrier()
        pltpu.sync_copy(idx_hbm.at[wid], idx_v)

        def grp(g, carry):
            pltpu.sync_copy(msg_hbm.at[pl.ds(wid * pt + g * bufrows, bufrows)], buf)
            cps = [
                pltpu.async_copy(
                    buf.at[pl.ds(t * C, C)],
                    acc.at[idx_v.at[g * k + t]],
                    sem,
                    add=True,
                )
                for t in range(k)
            ]
            for cp in cps:
                cp.wait()
            return carry

        lax.fori_loop(0, chunks // k, grp, 0)
        plsc.subcore_barrier()

        @pl.when(sid == 0)
        def _():
            pltpu.sync_copy(acc, out_hbm.at[cid])

    return scatter


def _combine_body(nd, final, p_ref, hd_ref, root_ref, bias_ref, wr, br, o_ref):
    p = p_ref[...]
    a = p[0, :nd] + p[1, :nd]
    agg = a[:, :HW] / jnp.maximum(a[:, HW:HW + 1], 1.0)
    o = agg + jnp.dot(hd_ref[...], root_ref[...],
                      preferred_element_type=jnp.float32) + bias_ref[...]
    h = _leaky(o)
    if final:
        o_ref[...] = _leaky(jnp.dot(h, wr[...], preferred_element_type=jnp.float32)
                            + br[...])
    else:
        o_ref[...] = h


def _combine(part, h_all, nd, root, bias, lin=None):
    full = lambda r: pl.BlockSpec(r, lambda i: (0,) * len(r))
    if lin is None:
        wr, br = jnp.zeros((16, 1), jnp.float32), jnp.zeros((1, 1), jnp.float32)
        out_w = HW
    else:
        wr, br = lin[0], lin[1].reshape(1, -1)
        out_w = lin[0].shape[1]
    args = [part, h_all, root, bias.reshape(1, -1), wr, br]
    specs = [full(part.shape), pl.BlockSpec((nd, HW), lambda i: (0, 0)),
             full(root.shape), full((1, 16)), full(wr.shape), full(br.shape)]
    return pl.pallas_call(
        functools.partial(_combine_body, nd, lin is not None),
        grid=(1,),
        in_specs=specs,
        out_specs=full((nd, out_w)),
        out_shape=jax.ShapeDtypeStruct((nd, out_w), jnp.float32),
    )(*args)


def kernel(x_target, x_other, e_feat0, e_feat1, params, edge_index0,
           edge_index1, h_id_target, h_id_other):
    p = params
    n_tgt = x_target.shape[0]

    xs = jnp.concatenate([x_target, x_other], axis=0)
    h0 = _embed(xs, p['emb_target'], p['emb_other'], blk=1000, ntb=n_tgt // 1000)

    def layer(h_all, nd, ei, ef, edge_ps, root, bias, k, lin=None):
        e = ef.shape[0]
        ep = -(-e // (NW * C)) * (NW * C)
        nda = nd + 8
        nblk = ep // EBLK
        src = jnp.concatenate(
            [ei[0].astype(jnp.int32), jnp.zeros((ep - e,), jnp.int32)])
        src = src.reshape(nblk, 8, EBLK // 8)
        src = src.transpose(0, 2, 1).reshape(NW, (ep // NW) // C, C)
        dst = jnp.concatenate(
            [ei[1].astype(jnp.int32), jnp.full((ep - e,), nd, jnp.int32)])
        dst = dst.reshape(nblk, 4, EBLK // 4)
        dst = dst.transpose(0, 2, 1).reshape(NW, (ep // NW) // C, C)
        eft = jnp.concatenate(
            [ef.T, jnp.zeros((HW, ep - e), jnp.float32)], axis=1)
        xj = _make_gather(ep, k)(h_all, src)
        xjp = xj.reshape(ep // 8, 128)
        msgp = _msg(eft, xjp, _edge_consts(edge_ps), blk=EBLK)
        msg_lin = msgp.reshape(ep, MW)
        zer = jnp.zeros((nda, MW), jnp.float32)
        part = _make_scatter(ep, nda, k)(msg_lin, dst, zer)
        return _combine(part, h_all[:nd], nd, root, bias, lin=lin)

    h1 = layer(h0, 5000, edge_index0, e_feat0, p['edge_nn0'], p['root0'],
               p['bias0'], k=5)
    out = layer(h1, 2000, edge_index1, e_feat1, p['edge_nn1'], p['root1'],
                p['bias1'], k=5, lin=p['lin1'])
    return out


# speedup vs baseline: 1.1036x; 1.1036x over previous
import functools

import numpy as np
import jax
import jax.numpy as jnp
from jax import lax
from jax.experimental import pallas as pl
from jax.experimental.pallas import tpu as pltpu
from jax.experimental.pallas import tpu_sc as plsc

NC = 2
NS = 16
NW = NC * NS
C = 100
HW = 16
MW = 32
EBLK = 3200


def _leaky(x):
    return jnp.where(x >= 0, x, 0.01 * x)


def _embed_body(ntb, x_ref, w0r, b0r, w1r, b1r, w2r, b2r, o_ref):
    sel = pl.program_id(0) < ntb
    w0 = jnp.where(sel, w0r[0], w0r[1])
    b0 = jnp.where(sel, b0r[0], b0r[1])
    w1 = jnp.where(sel, w1r[0], w1r[1])
    b1 = jnp.where(sel, b1r[0], b1r[1])
    w2 = jnp.where(sel, w2r[0], w2r[1])
    b2 = jnp.where(sel, b2r[0], b2r[1])
    x = x_ref[...]
    h = _leaky(jnp.dot(x, w0, preferred_element_type=jnp.float32) + b0[None, :])
    h = _leaky(jnp.dot(h, w1, preferred_element_type=jnp.float32) + b1[None, :])
    h = jnp.dot(h, w2, preferred_element_type=jnp.float32) + b2[None, :]
    o_ref[...] = h


def _embed(xs, wt, wo, blk, ntb):
    n = xs.shape[0]
    ws = [jnp.stack([a[0], b[0]]) for a, b in zip(wt, wo)]
    bs = [jnp.stack([a[1], b[1]]) for a, b in zip(wt, wo)]
    full = lambda r: pl.BlockSpec(r, lambda i: (0,) * len(r))
    args = []
    specs = [pl.BlockSpec((blk, xs.shape[1]), lambda i: (i, 0))]
    for w, b in zip(ws, bs):
        args += [w, b]
        specs += [full(w.shape), full(b.shape)]
    return pl.pallas_call(
        functools.partial(_embed_body, ntb),
        grid=(n // blk,),
        in_specs=specs,
        out_specs=pl.BlockSpec((blk, HW), lambda i: (i, 0)),
        out_shape=jax.ShapeDtypeStruct((n, HW), jnp.float32),
    )(xs, *args)


def _make_gather(e, k):
    pt = e // NW
    chunks = pt // C
    mesh = plsc.VectorSubcoreMesh(core_axis_name="c", subcore_axis_name="s")

    @functools.partial(
        pl.kernel,
        out_type=jax.ShapeDtypeStruct((e, HW), jnp.float32),
        mesh=mesh,
        compiler_params=pltpu.CompilerParams(use_tc_tiling_on_sc=False),
        scratch_types=[
            pltpu.VMEM((chunks, C), jnp.int32),
            pltpu.VMEM((k, C, HW), jnp.float32),
            pltpu.SemaphoreType.DMA,
            pltpu.SemaphoreType.DMA,
        ],
    )
    def gather(h_hbm, idx_hbm, out_hbm, idx_v, bufs, gsem, osem):
        wid = lax.axis_index("s") * NC + lax.axis_index("c")
        pltpu.sync_copy(idx_hbm.at[wid], idx_v)

        def grp(g, carry):
            cps = [
                pltpu.async_copy(h_hbm.at[idx_v.at[g * k + t]], bufs.at[t], gsem)
                for t in range(k)
            ]
            for cp in cps:
                cp.wait()
            ocs = [
                pltpu.async_copy(
                    bufs.at[t],
                    out_hbm.at[pl.ds((wid * chunks + g * k + t) * C, C)],
                    osem,
                )
                for t in range(k)
            ]
            for oc in ocs:
                oc.wait()
            return carry

        lax.fori_loop(0, chunks // k, grp, 0)

    return gather


def _msg_body(blk, eft_ref, xjp_ref, w1t, b1c, w2t, b2c, t2hi, t2lo, b3t,
              o_ref):
    m = blk // 8
    ef_t = eft_ref[...]
    xjT = xjp_ref[...].T
    h1 = _leaky(jnp.dot(w1t[...], ef_t, preferred_element_type=jnp.float32)
                + b1c[...])
    h2 = _leaky(jnp.dot(w2t[...], h1, preferred_element_type=jnp.float32)
                + b2c[...])
    msg_t = []
    for t in range(8):
        xjt = xjT[t * HW:(t + 1) * HW]
        h2t = h2[:, t * m:(t + 1) * m]
        z = jnp.concatenate([xjt[i:i + 1, :] * h2t for i in range(HW)], axis=0)
        zhi = z.astype(jnp.bfloat16).astype(jnp.float32)
        zlo = z - zhi
        mt = (jnp.dot(t2hi[...], zhi, preferred_element_type=jnp.float32)
              + jnp.dot(t2hi[...], zlo, preferred_element_type=jnp.float32)
              + jnp.dot(t2lo[...], zhi, preferred_element_type=jnp.float32)
              + jnp.dot(b3t[...], xjt, preferred_element_type=jnp.float32))
        msg_t.append(mt)
    cnt = jnp.concatenate(
        [jnp.ones((1, m), jnp.float32), jnp.zeros((HW - 1, m), jnp.float32)],
        axis=0)
    low = jnp.concatenate(
        [msg_t[0], cnt, msg_t[2], cnt, msg_t[4], cnt, msg_t[6], cnt], axis=0)
    high = jnp.concatenate(
        [msg_t[1], cnt, msg_t[3], cnt, msg_t[5], cnt, msg_t[7], cnt], axis=0)
    o_ref[...] = jnp.concatenate([low, high], axis=1).T


def _edge_consts(edge_ps):
    (w1, b1), (w2, b2), (w3, b3) = edge_ps
    hdim = w3.shape[0]
    out_c = 16
    in_c = w3.shape[1] // out_c
    t2 = w3.reshape(hdim, in_c, out_c).transpose(1, 0, 2).reshape(in_c * hdim, out_c)
    b3r = b3.reshape(in_c, out_c)
    t2t = t2.T
    t2hi = t2t.astype(jnp.bfloat16).astype(jnp.float32)
    t2lo = t2t - t2hi
    return (w1.T, b1.reshape(-1, 1), w2.T, b2.reshape(-1, 1), t2hi, t2lo,
            b3r.T)


def _msg(ef, xjp, consts, blk):
    e = ef.shape[0]
    eft = ef.T
    full = lambda r: pl.BlockSpec(r, lambda i: (0,) * len(r))
    specs = [pl.BlockSpec((16, blk), lambda i: (0, i)),
             pl.BlockSpec((blk // 8, 128), lambda i: (i, 0))]
    specs += [full(c.shape) for c in consts]
    return pl.pallas_call(
        functools.partial(_msg_body, blk),
        grid=(e // blk,),
        in_specs=specs,
        out_specs=pl.BlockSpec((blk // 4, 128), lambda i: (i, 0)),
        out_shape=jax.ShapeDtypeStruct((e // 4, 128), jnp.float32),
    )(eft, xjp, *consts)


def _make_scatter(e, nd, k):
    pt = e // NW
    chunks = pt // C
    bufrows = k * C
    mesh = plsc.VectorSubcoreMesh(core_axis_name="c", subcore_axis_name="s")

    @functools.partial(
        pl.kernel,
        out_type=jax.ShapeDtypeStruct((NC, nd, MW), jnp.float32),
        mesh=mesh,
        compiler_params=pltpu.CompilerParams(use_tc_tiling_on_sc=False),
        scratch_types=[
            pltpu.VMEM((chunks, C), jnp.int32),
            pltpu.VMEM((bufrows, MW), jnp.float32),
            pltpu.VMEM_SHARED((nd, MW), jnp.float32),
            pltpu.SemaphoreType.DMA,
        ],
    )
    def scatter(msg_hbm, idx_hbm, zer_hbm, out_hbm, idx_v, buf, acc, sem):
        cid = lax.axis_index("c")
        sid = lax.axis_index("s")
        wid = sid * NC + cid

        @pl.when(sid == 0)
        def _():
            pltpu.sync_copy(zer_hbm, acc)

        plsc.subcore_barrier()
        pltpu.sync_copy(idx_hbm.at[wid], idx_v)

        def grp(g, carry):
            pltpu.sync_copy(msg_hbm.at[pl.ds(wid * pt + g * bufrows, bufrows)], buf)
            cps = [
                pltpu.async_copy(
                    buf.at[pl.ds(t * C, C)],
                    acc.at[idx_v.at[g * k + t]],
                    sem,
                    add=True,
                )
                for t in range(k)
            ]
            for cp in cps:
                cp.wait()
            return carry

        lax.fori_loop(0, chunks // k, grp, 0)
        plsc.subcore_barrier()

        @pl.when(sid == 0)
        def _():
            pltpu.sync_copy(acc, out_hbm.at[cid])

    return scatter


def _combine_body(final, p_ref, hd_ref, root_ref, bias_ref, wr, br, o_ref):
    p = p_ref[...]
    a = p[0] + p[1]
    agg = a[:, :HW] / jnp.maximum(a[:, HW:HW + 1], 1.0)
    o = agg + jnp.dot(hd_ref[...], root_ref[...],
                      preferred_element_type=jnp.float32) + bias_ref[...]
    h = _leaky(o)
    if final:
        o_ref[...] = _leaky(jnp.dot(h, wr[...], preferred_element_type=jnp.float32)
                            + br[...])
    else:
        o_ref[...] = h


def _combine(part, h_all, nd, root, bias, lin=None):
    full = lambda r: pl.BlockSpec(r, lambda i: (0,) * len(r))
    if lin is None:
        wr, br = jnp.zeros((16, 1), jnp.float32), jnp.zeros((1, 1), jnp.float32)
        out_w = HW
    else:
        wr, br = lin[0], lin[1].reshape(1, -1)
        out_w = lin[0].shape[1]
    args = [part, h_all, root, bias.reshape(1, -1), wr, br]
    specs = [full(part.shape), pl.BlockSpec((nd, HW), lambda i: (0, 0)),
             full(root.shape), full((1, 16)), full(wr.shape), full(br.shape)]
    return pl.pallas_call(
        functools.partial(_combine_body, lin is not None),
        grid=(1,),
        in_specs=specs,
        out_specs=full((nd, out_w)),
        out_shape=jax.ShapeDtypeStruct((nd, out_w), jnp.float32),
    )(*args)


def kernel(x_target, x_other, e_feat0, e_feat1, params, edge_index0,
           edge_index1, h_id_target, h_id_other):
    p = params
    n_tgt = x_target.shape[0]

    xs = jnp.concatenate([x_target, x_other], axis=0)
    h0 = _embed(xs, p['emb_target'], p['emb_other'], blk=1000, ntb=n_tgt // 1000)

    def layer(h_all, nd, ei, ef, edge_ps, root, bias, k, lin=None):
        e = ef.shape[0]
        nblk = e // EBLK
        src = ei[0].astype(jnp.int32).reshape(nblk, 8, EBLK // 8)
        src = src.transpose(0, 2, 1).reshape(NW, (e // NW) // C, C)
        dst = ei[1].astype(jnp.int32).reshape(nblk, 4, EBLK // 4)
        dst = dst.transpose(0, 2, 1).reshape(NW, (e // NW) // C, C)
        xj = _make_gather(e, k)(h_all, src)
        xjp = xj.reshape(e // 8, 128)
        msgp = _msg(ef, xjp, _edge_consts(edge_ps), blk=EBLK)
        msg_lin = msgp.reshape(e, MW)
        zer = jnp.zeros((nd, MW), jnp.float32)
        part = _make_scatter(e, nd, k)(msg_lin, dst, zer)
        return _combine(part, h_all[:nd], nd, root, bias, lin=lin)

    h1 = layer(h0, 5000, edge_index0, e_feat0, p['edge_nn0'], p['root0'],
               p['bias0'], k=10)
    out = layer(h1, 2000, edge_index1, e_feat1, p['edge_nn1'], p['root1'],
                p['bias1'], k=5, lin=p['lin1'])
    return out
